# Initial kernel scaffold; baseline (speedup 1.0000x reference)
import jax
import jax.numpy as jnp
from jax.experimental import pallas as pl


def kernel(x, Wqkv, Wo, Ww, bw):
    raise NotImplementedError("write your pallas kernel here")


# speedup vs baseline: 3.5836x; 3.5836x over previous
import numpy as np
import jax
import jax.numpy as jnp
from jax import lax
from jax.experimental import pallas as pl

_K = 16
_H = 8


def _smha_kernel(xb_ref, x_ref, wqkvT_ref, woT_ref, wcol_ref, bw_ref, y_ref):
    N, D = x_ref.shape[1], x_ref.shape[2]
    R = xb_ref.shape[1]
    K, H = _K, _H
    dh = D // H
    f32 = jnp.float32

    x_all = x_ref[0]
    xb = xb_ref[0]

    xsq = x_all * x_all
    st_row = lax.dot_general(jnp.ones((1, D), f32), xsq,
                             (((1,), (1,)), ((), ())),
                             precision=lax.Precision.HIGHEST,
                             preferred_element_type=f32)
    xdot = lax.dot_general(xb.astype(jnp.bfloat16), x_all.astype(jnp.bfloat16),
                           (((1,), (1,)), ((), ())),
                           preferred_element_type=f32)
    cur = st_row - 2.0 * xdot
    col = lax.broadcasted_iota(jnp.int32, (R, N), 1)
    ams = []
    for _ in range(K):
        m = jnp.min(cur, axis=1, keepdims=True)
        am = jnp.min(jnp.where(cur <= m, col, N), axis=1, keepdims=True)
        ams.append(am)
        cur = jnp.where(col == am, jnp.float32(jnp.inf), cur)

    wqkvT = wqkvT_ref[...]
    q_list, k_list, v_list = [], [], []
    for b in range(K):
        oh = (col == ams[b]).astype(f32)
        xg = jnp.dot(oh, x_all, preferred_element_type=f32)
        qkv = jnp.dot(xg, wqkvT, preferred_element_type=f32)
        q_list.append(qkv[:, :D].reshape(R, 1, D))
        k_list.append(qkv[:, D:2 * D].reshape(R, 1, D))
        v_list.append(qkv[:, 2 * D:].reshape(R, 1, D))
    Ks = jnp.concatenate(k_list, axis=1).reshape(R * K, D)
    Vs = jnp.concatenate(v_list, axis=1).reshape(R * K, D)

    di = lax.broadcasted_iota(jnp.int32, (D, H), 0)
    hi = lax.broadcasted_iota(jnp.int32, (D, H), 1)
    G = jnp.where(di // dh == hi, f32(1.0 / np.sqrt(dh)), f32(0.0))
    hi2 = lax.broadcasted_iota(jnp.int32, (H, D), 0)
    di2 = lax.broadcasted_iota(jnp.int32, (H, D), 1)
    GT = jnp.where(di2 // dh == hi2, f32(1.0), f32(0.0))

    s_cols = []
    for a in range(K):
        qa_exp = jnp.broadcast_to(q_list[a], (R, K, D)).reshape(R * K, D)
        prod = qa_exp * Ks
        s_cols.append(jnp.dot(prod, G, preferred_element_type=f32))
    S3 = jnp.concatenate(s_cols, axis=1).reshape(R, K, K * H)
    mx = jnp.max(S3, axis=1, keepdims=True)
    E3 = jnp.exp(S3 - mx)
    sm = jnp.sum(E3, axis=1, keepdims=True)
    attn = (E3 / sm).reshape(R * K, K * H)

    ci = lax.broadcasted_iota(jnp.int32, (K * H, K), 0)
    ai = lax.broadcasted_iota(jnp.int32, (K * H, K), 1)
    Mavg = jnp.where(ci // H == ai, f32(1.0 / H), f32(0.0))
    wav = jnp.dot(attn, Mavg, preferred_element_type=f32)
    z = jnp.sum((wav * wcol_ref[...]).reshape(R, K, K), axis=1)
    s = 1.0 / (1.0 + jnp.exp(-(z + bw_ref[0, 0])))
    aw = s / jnp.sum(s, axis=1, keepdims=True)

    acc = jnp.zeros((R, D), f32)
    for a in range(K):
        w_exp = jnp.dot(attn[:, a * H:(a + 1) * H], GT,
                        preferred_element_type=f32)
        out_a = jnp.sum((w_exp * Vs).reshape(R, K, D), axis=1)
        acc = acc + out_a * aw[:, a:a + 1]

    y = jnp.dot(acc, woT_ref[...], preferred_element_type=f32)
    y_ref[0] = y


def kernel(x, Wqkv, Wo, Ww, bw):
    B, N, D = x.shape
    R = 256 if N % 256 == 0 else N
    grid = (B, N // R)
    wqkvT = Wqkv.T
    woT = Wo.T
    wcol = jnp.tile(Ww.reshape(_K, 1), (R, 1))
    bwa = bw.reshape(1, 1)
    return pl.pallas_call(
        _smha_kernel,
        grid=grid,
        in_specs=[
            pl.BlockSpec((1, R, D), lambda b, r: (b, r, 0)),
            pl.BlockSpec((1, N, D), lambda b, r: (b, 0, 0)),
            pl.BlockSpec((D, 3 * D), lambda b, r: (0, 0)),
            pl.BlockSpec((D, D), lambda b, r: (0, 0)),
            pl.BlockSpec((R * _K, 1), lambda b, r: (0, 0)),
            pl.BlockSpec((1, 1), lambda b, r: (0, 0)),
        ],
        out_specs=pl.BlockSpec((1, R, D), lambda b, r: (b, r, 0)),
        out_shape=jax.ShapeDtypeStruct((B, N, D), jnp.float32),
    )(x, x, wqkvT, woT, wcol, bwa)


# speedup vs baseline: 3.6063x; 1.0063x over previous
import numpy as np
import jax
import jax.numpy as jnp
from jax import lax
from jax.experimental import pallas as pl

_K = 16
_H = 8


def _smha_kernel(xb_ref, x_ref, wqkvT_ref, woT_ref, wcol_ref, bw_ref, y_ref):
    N, D = x_ref.shape[1], x_ref.shape[2]
    R = xb_ref.shape[1]
    K, H = _K, _H
    dh = D // H
    f32 = jnp.float32

    x_all = x_ref[0]
    xb = xb_ref[0]

    xsq = x_all * x_all
    st_row = lax.dot_general(jnp.ones((1, D), f32), xsq,
                             (((1,), (1,)), ((), ())),
                             precision=lax.Precision.HIGHEST,
                             preferred_element_type=f32)
    xdot = lax.dot_general(xb.astype(jnp.bfloat16), x_all.astype(jnp.bfloat16),
                           (((1,), (1,)), ((), ())),
                           preferred_element_type=f32)
    cur = st_row - 2.0 * xdot
    col = lax.broadcasted_iota(jnp.int32, (R, N), 1)
    ams = []
    for _ in range(K):
        m = jnp.min(cur, axis=1, keepdims=True)
        am = jnp.min(jnp.where(cur <= m, col, N), axis=1, keepdims=True)
        ams.append(am)
        cur = jnp.where(col == am, jnp.float32(jnp.inf), cur)

    bf16 = jnp.bfloat16
    x16 = x_all.astype(bf16)
    wqkvT = wqkvT_ref[...].astype(bf16)
    q_list, k_list, v_list = [], [], []
    for b in range(K):
        oh = (col == ams[b]).astype(f32).astype(bf16)
        xg = jnp.dot(oh, x16, preferred_element_type=f32)
        qkv = jnp.dot(xg.astype(bf16), wqkvT,
                      preferred_element_type=f32)
        q_list.append(qkv[:, :D].reshape(R, 1, D))
        k_list.append(qkv[:, D:2 * D].reshape(R, 1, D))
        v_list.append(qkv[:, 2 * D:].reshape(R, 1, D))
    Ks = jnp.concatenate(k_list, axis=1).reshape(R * K, D)
    Vs = jnp.concatenate(v_list, axis=1).reshape(R * K, D)

    di = lax.broadcasted_iota(jnp.int32, (D, H), 0)
    hi = lax.broadcasted_iota(jnp.int32, (D, H), 1)
    G = jnp.where(di // dh == hi, f32(1.0), f32(0.0)).astype(bf16)
    hi2 = lax.broadcasted_iota(jnp.int32, (H, D), 0)
    di2 = lax.broadcasted_iota(jnp.int32, (H, D), 1)
    GT = jnp.where(di2 // dh == hi2, f32(1.0), f32(0.0)).astype(bf16)
    inv_sqrt_dh = f32(1.0 / np.sqrt(dh))

    s_cols = []
    for a in range(K):
        qa_exp = jnp.broadcast_to(q_list[a], (R, K, D)).reshape(R * K, D)
        prod = (qa_exp * Ks).astype(bf16)
        s_cols.append(jnp.dot(prod, G,
                              preferred_element_type=f32) * inv_sqrt_dh)
    S3 = jnp.concatenate(s_cols, axis=1).reshape(R, K, K * H)
    mx = jnp.max(S3, axis=1, keepdims=True)
    E3 = jnp.exp(S3 - mx)
    sm = jnp.sum(E3, axis=1, keepdims=True)
    attn = (E3 / sm).reshape(R * K, K * H)

    ci = lax.broadcasted_iota(jnp.int32, (K * H, K), 0)
    ai = lax.broadcasted_iota(jnp.int32, (K * H, K), 1)
    Mavg = jnp.where(ci // H == ai, f32(1.0 / H), f32(0.0)).astype(bf16)
    wav = jnp.dot(attn.astype(bf16), Mavg,
                  preferred_element_type=f32)
    z = jnp.sum((wav * wcol_ref[...]).reshape(R, K, K), axis=1)
    s = 1.0 / (1.0 + jnp.exp(-(z + bw_ref[0, 0])))
    aw = s / jnp.sum(s, axis=1, keepdims=True)

    acc = jnp.zeros((R, D), f32)
    for a in range(K):
        w_exp = jnp.dot(attn[:, a * H:(a + 1) * H].astype(bf16), GT,
                        preferred_element_type=f32)
        out_a = jnp.sum((w_exp * Vs).reshape(R, K, D), axis=1)
        acc = acc + out_a * aw[:, a:a + 1]

    y = jnp.dot(acc.astype(bf16), woT_ref[...].astype(bf16),
                preferred_element_type=f32)
    y_ref[0] = y


def kernel(x, Wqkv, Wo, Ww, bw):
    B, N, D = x.shape
    R = 256 if N % 256 == 0 else N
    grid = (B, N // R)
    wqkvT = Wqkv.T
    woT = Wo.T
    wcol = jnp.tile(Ww.reshape(_K, 1), (R, 1))
    bwa = bw.reshape(1, 1)
    return pl.pallas_call(
        _smha_kernel,
        grid=grid,
        in_specs=[
            pl.BlockSpec((1, R, D), lambda b, r: (b, r, 0)),
            pl.BlockSpec((1, N, D), lambda b, r: (b, 0, 0)),
            pl.BlockSpec((D, 3 * D), lambda b, r: (0, 0)),
            pl.BlockSpec((D, D), lambda b, r: (0, 0)),
            pl.BlockSpec((R * _K, 1), lambda b, r: (0, 0)),
            pl.BlockSpec((1, 1), lambda b, r: (0, 0)),
        ],
        out_specs=pl.BlockSpec((1, R, D), lambda b, r: (b, r, 0)),
        out_shape=jax.ShapeDtypeStruct((B, N, D), jnp.float32),
    )(x, x, wqkvT, woT, wcol, bwa)


# speedup vs baseline: 4.8785x; 1.3528x over previous
import functools
import numpy as np
import jax
import jax.numpy as jnp
from jax import lax
from jax.experimental import pallas as pl
from jax.experimental.pallas import tpu as pltpu
from jax.experimental.pallas import tpu_sc as plsc

_K = 16
_H = 8


def _knn_kernel(xb_ref, x_ref, idx_ref):
    N, D = x_ref.shape[1], x_ref.shape[2]
    R = xb_ref.shape[1]
    f32 = jnp.float32
    x_all = x_ref[0]
    xb = xb_ref[0]
    xsq = x_all * x_all
    st_row = lax.dot_general(jnp.ones((1, D), f32), xsq,
                             (((1,), (1,)), ((), ())),
                             precision=lax.Precision.HIGHEST,
                             preferred_element_type=f32)
    xdot = lax.dot_general(xb.astype(jnp.bfloat16), x_all.astype(jnp.bfloat16),
                           (((1,), (1,)), ((), ())),
                           preferred_element_type=f32)
    cur = st_row - 2.0 * xdot
    col = lax.broadcasted_iota(jnp.int32, (R, N), 1)
    ams = []
    for _ in range(_K):
        m = jnp.min(cur, axis=1, keepdims=True)
        am = jnp.min(jnp.where(cur <= m, col, N), axis=1, keepdims=True)
        ams.append(am)
        cur = jnp.where(col == am, jnp.float32(jnp.inf), cur)
    base = pl.program_id(0) * N
    idx_ref[0] = jnp.concatenate(ams, axis=1) + base


def _sc_gather(x_hbm, idx_hbm, out_hbm, idx_v, rows_v, sem):
    nc = 2
    wid = lax.axis_index("s") * nc + lax.axis_index("c")
    total = idx_hbm.shape[0]
    chunk = idx_v.shape[0]
    per_w = total // (nc * 16)
    steps = per_w // chunk
    base = wid * per_w

    def body(g, _):
        start = base + g * chunk
        pltpu.sync_copy(idx_hbm.at[pl.ds(start, chunk)], idx_v)
        pltpu.async_copy(x_hbm.at[idx_v], rows_v, sem).wait()
        pltpu.sync_copy(rows_v, out_hbm.at[pl.ds(start, chunk)])
        return 0

    lax.fori_loop(0, steps, body, 0)


def _attn_kernel(xg_ref, wqkvT_ref, woT_ref, wcol_ref, bw_ref, y_ref):
    RK, D = xg_ref.shape
    K, H = _K, _H
    R = RK // K
    dh = D // H
    f32 = jnp.float32
    bf16 = jnp.bfloat16

    wqkvT = wqkvT_ref[...].astype(bf16)
    qkv = jnp.dot(xg_ref[...].astype(bf16), wqkvT,
                  preferred_element_type=f32)
    qkv3 = qkv.reshape(R, K, 3 * D)
    Ks = qkv[:, D:2 * D]
    Vs = qkv[:, 2 * D:]

    di = lax.broadcasted_iota(jnp.int32, (D, H), 0)
    hi = lax.broadcasted_iota(jnp.int32, (D, H), 1)
    G = jnp.where(di // dh == hi, f32(1.0), f32(0.0)).astype(bf16)
    hi2 = lax.broadcasted_iota(jnp.int32, (H, D), 0)
    di2 = lax.broadcasted_iota(jnp.int32, (H, D), 1)
    GT = jnp.where(di2 // dh == hi2, f32(1.0), f32(0.0)).astype(bf16)
    inv_sqrt_dh = f32(1.0 / np.sqrt(dh))

    s_cols = []
    for a in range(K):
        qa = qkv3[:, a, :D].reshape(R, 1, D)
        qa_exp = jnp.broadcast_to(qa, (R, K, D)).reshape(RK, D)
        prod = (qa_exp * Ks).astype(bf16)
        s_cols.append(jnp.dot(prod, G,
                              preferred_element_type=f32) * inv_sqrt_dh)
    S3 = jnp.concatenate(s_cols, axis=1).reshape(R, K, K * H)
    mx = jnp.max(S3, axis=1, keepdims=True)
    E3 = jnp.exp(S3 - mx)
    sm = jnp.sum(E3, axis=1, keepdims=True)
    attn = (E3 / sm).reshape(RK, K * H)

    ci = lax.broadcasted_iota(jnp.int32, (K * H, K), 0)
    ai = lax.broadcasted_iota(jnp.int32, (K * H, K), 1)
    Mavg = jnp.where(ci // H == ai, f32(1.0 / H), f32(0.0)).astype(bf16)
    wav = jnp.dot(attn.astype(bf16), Mavg,
                  preferred_element_type=f32)
    z = jnp.sum((wav * wcol_ref[...]).reshape(R, K, K), axis=1)
    s = 1.0 / (1.0 + jnp.exp(-(z + bw_ref[0, 0])))
    aw = s / jnp.sum(s, axis=1, keepdims=True)

    acc = jnp.zeros((R, D), f32)
    for a in range(K):
        w_exp = jnp.dot(attn[:, a * H:(a + 1) * H].astype(bf16), GT,
                        preferred_element_type=f32)
        out_a = jnp.sum((w_exp * Vs).reshape(R, K, D), axis=1)
        acc = acc + out_a * aw[:, a:a + 1]

    y_ref[...] = jnp.dot(acc.astype(bf16), woT_ref[...].astype(bf16),
                         preferred_element_type=f32)


def kernel(x, Wqkv, Wo, Ww, bw):
    B, N, D = x.shape
    K = _K
    R = 256 if N % 256 == 0 else N
    M = B * N

    idxf = pl.pallas_call(
        _knn_kernel,
        grid=(B, N // R),
        in_specs=[
            pl.BlockSpec((1, R, D), lambda b, r: (b, r, 0)),
            pl.BlockSpec((1, N, D), lambda b, r: (b, 0, 0)),
        ],
        out_specs=pl.BlockSpec((1, R, K), lambda b, r: (b, r, 0)),
        out_shape=jax.ShapeDtypeStruct((B, N, K), jnp.int32),
    )(x, x)

    x2d = x.reshape(M, D)
    idx1 = idxf.reshape(M * K)
    chunk = 128

    sc_gather = functools.partial(
        pl.kernel,
        out_type=jax.ShapeDtypeStruct((M * K, D), jnp.float32),
        mesh=plsc.VectorSubcoreMesh(core_axis_name="c", subcore_axis_name="s"),
        scratch_types=[
            pltpu.VMEM((chunk,), jnp.int32),
            pltpu.VMEM((chunk, D), jnp.float32),
            pltpu.SemaphoreType.DMA,
        ],
    )(_sc_gather)
    xg2d = sc_gather(x2d, idx1)

    wqkvT = Wqkv.T
    woT = Wo.T
    ra = 128 if M % 128 == 0 else N
    wcol = jnp.tile(Ww.reshape(K, 1), (ra, 1))
    bwa = bw.reshape(1, 1)
    y2d = pl.pallas_call(
        _attn_kernel,
        grid=(M // ra,),
        in_specs=[
            pl.BlockSpec((ra * K, D), lambda r: (r, 0)),
            pl.BlockSpec((D, 3 * D), lambda r: (0, 0)),
            pl.BlockSpec((D, D), lambda r: (0, 0)),
            pl.BlockSpec((ra * K, 1), lambda r: (0, 0)),
            pl.BlockSpec((1, 1), lambda r: (0, 0)),
        ],
        out_specs=pl.BlockSpec((ra, D), lambda r: (r, 0)),
        out_shape=jax.ShapeDtypeStruct((M, D), jnp.float32),
    )(xg2d, wqkvT, woT, wcol, bwa)
    return y2d.reshape(B, N, D)


# speedup vs baseline: 5.2153x; 1.0690x over previous
import functools
import numpy as np
import jax
import jax.numpy as jnp
from jax import lax
from jax.experimental import pallas as pl
from jax.experimental.pallas import tpu as pltpu
from jax.experimental.pallas import tpu_sc as plsc

_K = 16
_H = 8


def _knn_kernel(xb_ref, x_ref, idx_ref):
    N, D = x_ref.shape[1], x_ref.shape[2]
    R = xb_ref.shape[1]
    f32 = jnp.float32
    x_all = x_ref[0]
    xb = xb_ref[0]
    xsq = x_all * x_all
    st_row = lax.dot_general(jnp.ones((1, D), f32), xsq,
                             (((1,), (1,)), ((), ())),
                             precision=lax.Precision.HIGHEST,
                             preferred_element_type=f32)
    xdot = lax.dot_general(xb.astype(jnp.bfloat16), x_all.astype(jnp.bfloat16),
                           (((1,), (1,)), ((), ())),
                           preferred_element_type=f32)
    cur = st_row - 2.0 * xdot
    col = lax.broadcasted_iota(jnp.int32, (R, N), 1)
    ams = []
    for _ in range(_K):
        m = jnp.min(cur, axis=1, keepdims=True)
        am = jnp.min(jnp.where(cur <= m, col, N), axis=1, keepdims=True)
        ams.append(am)
        cur = jnp.where(col == am, jnp.float32(jnp.inf), cur)
    base = pl.program_id(0) * N
    idx_ref[0] = jnp.concatenate(ams, axis=1) + base


def _sc_gather(x_hbm, idx_hbm, out_hbm, idx_v, rows_v, sem):
    nc = 2
    wid = lax.axis_index("s") * nc + lax.axis_index("c")
    total = idx_hbm.shape[0]
    chunk = idx_v.shape[0]
    per_w = total // (nc * 16)
    steps = per_w // chunk
    base = wid * per_w

    def body(g, _):
        start = base + g * chunk
        pltpu.sync_copy(idx_hbm.at[pl.ds(start, chunk)], idx_v)
        pltpu.async_copy(x_hbm.at[idx_v], rows_v, sem).wait()
        pltpu.sync_copy(rows_v, out_hbm.at[pl.ds(start, chunk)])
        return 0

    lax.fori_loop(0, steps, body, 0)


def _attn_kernel(xg_ref, wqkvT_ref, woT_ref, wcol_ref, bw_ref, y_ref):
    RK, D = xg_ref.shape
    K, H = _K, _H
    R = RK // K
    dh = D // H
    f32 = jnp.float32
    bf16 = jnp.bfloat16

    wqkvT = wqkvT_ref[...].astype(bf16)
    qkv = jnp.dot(xg_ref[...].astype(bf16), wqkvT,
                  preferred_element_type=f32)
    qkv3 = qkv.reshape(R, K, 3 * D)
    Ks = qkv[:, D:2 * D]
    Vs = qkv[:, 2 * D:]

    di = lax.broadcasted_iota(jnp.int32, (D, H), 0)
    hi = lax.broadcasted_iota(jnp.int32, (D, H), 1)
    G = jnp.where(di // dh == hi, f32(1.0), f32(0.0)).astype(bf16)
    hi2 = lax.broadcasted_iota(jnp.int32, (H, D), 0)
    di2 = lax.broadcasted_iota(jnp.int32, (H, D), 1)
    GT = jnp.where(di2 // dh == hi2, f32(1.0), f32(0.0)).astype(bf16)
    inv_sqrt_dh = f32(1.0 / np.sqrt(dh))

    s_cols = []
    for a in range(K):
        qa = qkv3[:, a, :D].reshape(R, 1, D)
        qa_exp = jnp.broadcast_to(qa, (R, K, D)).reshape(RK, D)
        prod = (qa_exp * Ks).astype(bf16)
        s_cols.append(jnp.dot(prod, G,
                              preferred_element_type=f32) * inv_sqrt_dh)
    S3 = jnp.concatenate(s_cols, axis=1).reshape(R, K, K * H)
    mx = jnp.max(S3, axis=1, keepdims=True)
    E3 = jnp.exp(S3 - mx)
    sm = jnp.sum(E3, axis=1, keepdims=True)
    attn = (E3 / sm).reshape(RK, K * H)

    ci = lax.broadcasted_iota(jnp.int32, (K * H, K), 0)
    ai = lax.broadcasted_iota(jnp.int32, (K * H, K), 1)
    Mavg = jnp.where(ci // H == ai, f32(1.0 / H), f32(0.0)).astype(bf16)
    wav = jnp.dot(attn.astype(bf16), Mavg,
                  preferred_element_type=f32)
    z = jnp.sum((wav * wcol_ref[...]).reshape(R, K, K), axis=1)
    s = 1.0 / (1.0 + jnp.exp(-(z + bw_ref[0, 0])))
    aw = s / jnp.sum(s, axis=1, keepdims=True)

    acc = jnp.zeros((R, D), f32)
    for a in range(K):
        w_exp = jnp.dot(attn[:, a * H:(a + 1) * H].astype(bf16), GT,
                        preferred_element_type=f32)
        out_a = jnp.sum((w_exp * Vs).reshape(R, K, D), axis=1)
        acc = acc + out_a * aw[:, a:a + 1]

    y_ref[...] = jnp.dot(acc.astype(bf16), woT_ref[...].astype(bf16),
                         preferred_element_type=f32)


def kernel(x, Wqkv, Wo, Ww, bw):
    B, N, D = x.shape
    K = _K
    R = 256 if N % 256 == 0 else N

    wqkvT = Wqkv.T
    woT = Wo.T
    ra = 128 if N % 128 == 0 else N
    wcol = jnp.tile(Ww.reshape(K, 1), (ra, 1))
    bwa = bw.reshape(1, 1)
    chunk = 128

    sc_gather = functools.partial(
        pl.kernel,
        out_type=jax.ShapeDtypeStruct((N * K, D), jnp.float32),
        mesh=plsc.VectorSubcoreMesh(core_axis_name="c", subcore_axis_name="s"),
        scratch_types=[
            pltpu.VMEM((chunk,), jnp.int32),
            pltpu.VMEM((chunk, D), jnp.float32),
            pltpu.SemaphoreType.DMA,
        ],
    )(_sc_gather)

    ys = []
    for b in range(B):
        xb3 = lax.slice(x, (b, 0, 0), (b + 1, N, D))
        idxf = pl.pallas_call(
            _knn_kernel,
            grid=(1, N // R),
            in_specs=[
                pl.BlockSpec((1, R, D), lambda g, r: (g, r, 0)),
                pl.BlockSpec((1, N, D), lambda g, r: (g, 0, 0)),
            ],
            out_specs=pl.BlockSpec((1, R, K), lambda g, r: (g, r, 0)),
            out_shape=jax.ShapeDtypeStruct((1, N, K), jnp.int32),
        )(xb3, xb3)
        xg2d = sc_gather(xb3.reshape(N, D), idxf.reshape(N * K))
        y2d = pl.pallas_call(
            _attn_kernel,
            grid=(N // ra,),
            in_specs=[
                pl.BlockSpec((ra * K, D), lambda r: (r, 0)),
                pl.BlockSpec((D, 3 * D), lambda r: (0, 0)),
                pl.BlockSpec((D, D), lambda r: (0, 0)),
                pl.BlockSpec((ra * K, 1), lambda r: (0, 0)),
                pl.BlockSpec((1, 1), lambda r: (0, 0)),
            ],
            out_specs=pl.BlockSpec((ra, D), lambda r: (r, 0)),
            out_shape=jax.ShapeDtypeStruct((N, D), jnp.float32),
        )(xg2d, wqkvT, woT, wcol, bwa)
        ys.append(y2d)
    return jnp.stack(ys, axis=0)
